# baseline (speedup 1.0000x reference)
import jax
import jax.numpy as jnp
from jax import lax
from jax.experimental import pallas as pl
from jax.experimental.pallas import tpu as pltpu
from jax.experimental.pallas import tpu_sc as plsc

_N = 10000
_D = 128
_H = 128
_C = 64
_E = 320000

_NC = 2
_NS = 16
_K = 128
_EP = 327680
_EPT = _EP // (_NC * _NS)
_CHUNKS = _EPT // _K
_SLOTS = 5
_GROUPS = _CHUNKS // _SLOTS
_TRASH = 128
_ACCR = _N + _TRASH
_RPT = 624
_TAIL0 = _RPT * _NS
_TAILN = _N - _TAIL0
_DW = 128

_mesh = plsc.VectorSubcoreMesh(core_axis_name="c", subcore_axis_name="s")


def _node_copy(sid, src, dst, src_base, dst_base):
    r0 = sid * _RPT
    pltpu.sync_copy(src.at[pl.ds(src_base + r0, _RPT)],
                    dst.at[pl.ds(dst_base + r0, _RPT)])

    @pl.when(sid == _NS - 1)
    def _():
        pltpu.sync_copy(src.at[pl.ds(src_base + _TAIL0, _TAILN)],
                        dst.at[pl.ds(dst_base + _TAIL0, _TAILN)])


def _deg_body(dst_hbm, z_hbm, ones_hbm, out_hbm, dv0, dv1, ones_v, acc_sh, si):
    cid = lax.axis_index("c")
    sid = lax.axis_index("s")
    e0 = (cid * _NS + sid) * _EPT
    dv = (dv0, dv1)
    pltpu.sync_copy(ones_hbm, ones_v)
    _node_copy(sid, z_hbm, acc_sh, 0, 0)
    plsc.subcore_barrier()

    def idx_copy(c, b):
        return pltpu.make_async_copy(
            dst_hbm.at[pl.ds(e0 + c * _K, _K)], dv[b], si.at[b])

    idx_copy(0, 0).start()
    idx_copy(1, 1).start()

    def body(p, carry):
        for b in range(2):
            c = 2 * p + b
            idx_copy(c, b).wait()
            pltpu.sync_copy(ones_v, acc_sh.at[dv[b]], add=True)

            @pl.when(p < _CHUNKS // 2 - 1)
            def _():
                idx_copy(c + 2, b).start()
        return carry

    lax.fori_loop(0, _CHUNKS // 2, body, 0)
    plsc.subcore_barrier()
    _node_copy(sid, acc_sh, out_hbm, 0, cid * _N)


_deg_call = pl.kernel(
    _deg_body,
    out_type=jax.ShapeDtypeStruct((2 * _N, _DW), jnp.float32),
    mesh=_mesh,
    scratch_types=[
        pltpu.VMEM((_K,), jnp.int32),
        pltpu.VMEM((_K,), jnp.int32),
        pltpu.VMEM((_K, _DW), jnp.float32),
        pltpu.VMEM_SHARED((_ACCR, _DW), jnp.float32),
        pltpu.SemaphoreType.DMA((2,)),
    ],
)


def _agg_body(h_hbm, z_hbm, src_hbm, dst_hbm, out_hbm,
              sv0, sv1, dv0, dv1, rows_v, acc_sh, si, sg):
    cid = lax.axis_index("c")
    sid = lax.axis_index("s")
    e0 = (cid * _NS + sid) * _EPT
    sv = (sv0, sv1)
    dv = (dv0, dv1)

    @pl.when(cid == 0)
    def _():
        _node_copy(sid, h_hbm, acc_sh, 0, 0)

    @pl.when(cid == 1)
    def _():
        _node_copy(sid, z_hbm, acc_sh, 0, 0)

    plsc.subcore_barrier()

    def src_copy(c, b):
        return pltpu.make_async_copy(
            src_hbm.at[pl.ds(e0 + c * _K, _K)], sv[b], si.at[b])

    def dst_copy(c, b):
        return pltpu.make_async_copy(
            dst_hbm.at[pl.ds(e0 + c * _K, _K)], dv[b], si.at[b])

    def body(c, carry):
        off = e0 + c * _K
        pltpu.sync_copy(src_hbm.at[pl.ds(off, _K)], sv0)
        pltpu.sync_copy(dst_hbm.at[pl.ds(off, _K)], dv0)
        pltpu.make_async_copy(h_hbm.at[sv0], rows_v, sg.at[0]).start()
        pltpu.make_async_copy(h_hbm.at[sv0], rows_v, sg.at[0]).wait()
        pltpu.sync_copy(rows_v, acc_sh.at[dv0], add=True)
        return carry

    lax.fori_loop(0, _CHUNKS, body, 0)

    plsc.subcore_barrier()
    _node_copy(sid, acc_sh, out_hbm, 0, cid * _N)


_agg_call = pl.kernel(
    _agg_body,
    out_type=jax.ShapeDtypeStruct((2 * _N, _H), jnp.float32),
    mesh=_mesh,
    scratch_types=(
        [pltpu.VMEM((_K,), jnp.int32) for _ in range(4)]
        + [pltpu.VMEM((_K, _H), jnp.float32),
           pltpu.VMEM_SHARED((_ACCR, _H), jnp.float32),
           pltpu.SemaphoreType.DMA((2,)),
           pltpu.SemaphoreType.DMA((1,))]
    ),
)

_R = 1000


def _rspec(w):
    return pl.BlockSpec((_R, w), lambda i: (i, 0))


def _fspec(r, c):
    return pl.BlockSpec((r, c), lambda i: (0, 0))


def _dinv_of(d0_ref, d1_ref):
    return lax.rsqrt(d0_ref[:, 0:1] + d1_ref[:, 0:1] + 1.0)


def _scale_body(d0_ref, d1_ref, x_ref, w_ref, o_ref):
    dinv = _dinv_of(d0_ref, d1_ref)
    m = jnp.dot(x_ref[:], w_ref[:], preferred_element_type=jnp.float32,
                precision=lax.Precision.HIGHEST)
    o_ref[:] = m * dinv


_scale_call = pl.pallas_call(
    _scale_body,
    grid=(_N // _R,),
    in_specs=[_rspec(_DW), _rspec(_DW), _rspec(_D), _fspec(_D, _H)],
    out_specs=_rspec(_H),
    out_shape=jax.ShapeDtypeStruct((_N, _H), jnp.float32),
)


def _mid_body(s0_ref, s1_ref, d0_ref, d1_ref, b1_ref, w2_ref, o_ref):
    dinv = _dinv_of(d0_ref, d1_ref)
    h1 = jnp.maximum((s0_ref[:] + s1_ref[:]) * dinv + b1_ref[:], 0.0)
    m = jnp.dot(h1, w2_ref[:], preferred_element_type=jnp.float32,
                precision=lax.Precision.HIGHEST)
    o_ref[:] = m * dinv


_mid_call = pl.pallas_call(
    _mid_body,
    grid=(_N // _R,),
    in_specs=[_rspec(_H), _rspec(_H), _rspec(_DW), _rspec(_DW),
              _fspec(1, _H), _fspec(_H, _H)],
    out_specs=_rspec(_H),
    out_shape=jax.ShapeDtypeStruct((_N, _H), jnp.float32),
)


def _fin_body(t0_ref, t1_ref, d0_ref, d1_ref, b2_ref, wc_ref, bc_ref, o_ref):
    dinv = _dinv_of(d0_ref, d1_ref)
    h2 = (t0_ref[:] + t1_ref[:]) * dinv + b2_ref[:]
    logits = jnp.dot(h2, wc_ref[:], preferred_element_type=jnp.float32,
                     precision=lax.Precision.HIGHEST) + bc_ref[:]
    m = jnp.max(logits, axis=1, keepdims=True)
    lse = jnp.log(jnp.sum(jnp.exp(logits - m), axis=1, keepdims=True)) + m
    o_ref[:] = logits - lse


_fin_call = pl.pallas_call(
    _fin_body,
    grid=(_N // _R,),
    in_specs=[_rspec(_H), _rspec(_H), _rspec(_DW), _rspec(_DW),
              _fspec(1, _H), _fspec(_H, _C), _fspec(1, _C)],
    out_specs=_rspec(_C),
    out_shape=jax.ShapeDtypeStruct((_N, _C), jnp.float32),
)


def kernel(x, edge_index, W1, b1, W2, b2, Wc, bc):
    npad = _EP - _E
    src = jnp.concatenate([edge_index[0], jnp.zeros((npad,), jnp.int32)])
    pad_dst = _N + jnp.arange(npad, dtype=jnp.int32) % _TRASH
    dst = jnp.concatenate([edge_index[1], pad_dst])
    z = jnp.zeros((_N, _H), jnp.float32)
    ones = jnp.ones((_K, _DW), jnp.float32)
    degs = _deg_call(dst, z, ones)
    d0, d1 = degs[:_N], degs[_N:]

    h1p = _scale_call(d0, d1, x, W1)
    s = _agg_call(h1p, z, src, dst)
    h2p = _mid_call(s[:_N], s[_N:], d0, d1, b1.reshape(1, _H), W2)
    t = _agg_call(h2p, z, src, dst)
    return _fin_call(t[:_N], t[_N:], d0, d1,
                     b2.reshape(1, _H), Wc, bc.reshape(1, _C))


# speedup vs baseline: 2.4809x; 2.4809x over previous
import jax
import jax.numpy as jnp
from jax import lax
from jax.experimental import pallas as pl
from jax.experimental.pallas import tpu as pltpu
from jax.experimental.pallas import tpu_sc as plsc

_N = 10000
_D = 128
_H = 128
_C = 64
_E = 320000

_NC = 2
_NS = 16
_K = 128
_EP = 327680
_EPT = _EP // (_NC * _NS)
_CHUNKS = _EPT // _K
_SLOTS = 5
_GROUPS = _CHUNKS // _SLOTS
_TRASH = 128
_ACCR = _N + _TRASH
_RPT = 624
_TAIL0 = _RPT * _NS
_TAILN = _N - _TAIL0
_DW = 128

_mesh = plsc.VectorSubcoreMesh(core_axis_name="c", subcore_axis_name="s")


def _node_copy(sid, src, dst, src_base, dst_base):
    r0 = sid * _RPT
    pltpu.sync_copy(src.at[pl.ds(src_base + r0, _RPT)],
                    dst.at[pl.ds(dst_base + r0, _RPT)])

    @pl.when(sid == _NS - 1)
    def _():
        pltpu.sync_copy(src.at[pl.ds(src_base + _TAIL0, _TAILN)],
                        dst.at[pl.ds(dst_base + _TAIL0, _TAILN)])


def _deg_body(dst_hbm, z_hbm, ones_hbm, out_hbm, dv0, dv1, ones_v, acc_sh, si):
    cid = lax.axis_index("c")
    sid = lax.axis_index("s")
    e0 = (cid * _NS + sid) * _EPT
    dv = (dv0, dv1)
    pltpu.sync_copy(ones_hbm, ones_v)
    _node_copy(sid, z_hbm, acc_sh, 0, 0)
    plsc.subcore_barrier()

    def idx_copy(c, b):
        return pltpu.make_async_copy(
            dst_hbm.at[pl.ds(e0 + c * _K, _K)], dv[b], si.at[b])

    idx_copy(0, 0).start()
    idx_copy(1, 1).start()

    def body(p, carry):
        for b in range(2):
            c = 2 * p + b
            idx_copy(c, b).wait()
            pltpu.sync_copy(ones_v, acc_sh.at[dv[b]], add=True)

            @pl.when(p < _CHUNKS // 2 - 1)
            def _():
                idx_copy(c + 2, b).start()
        return carry

    lax.fori_loop(0, _CHUNKS // 2, body, 0)
    plsc.subcore_barrier()
    _node_copy(sid, acc_sh, out_hbm, 0, cid * _N)


_deg_call = pl.kernel(
    _deg_body,
    out_type=jax.ShapeDtypeStruct((2 * _N, _DW), jnp.float32),
    mesh=_mesh,
    scratch_types=[
        pltpu.VMEM((_K,), jnp.int32),
        pltpu.VMEM((_K,), jnp.int32),
        pltpu.VMEM((_K, _DW), jnp.float32),
        pltpu.VMEM_SHARED((_ACCR, _DW), jnp.float32),
        pltpu.SemaphoreType.DMA((2,)),
    ],
)


def _agg_body(h_hbm, z_hbm, src_hbm, dst_hbm, out_hbm,
              sv0, sv1, dv0, dv1, rows_v, acc_sh, si, sg):
    cid = lax.axis_index("c")
    sid = lax.axis_index("s")
    e0 = (cid * _NS + sid) * _EPT
    sv = (sv0, sv1)
    dv = (dv0, dv1)

    @pl.when(cid == 0)
    def _():
        _node_copy(sid, h_hbm, acc_sh, 0, 0)

    @pl.when(cid == 1)
    def _():
        _node_copy(sid, z_hbm, acc_sh, 0, 0)

    plsc.subcore_barrier()

    def src_copy(c, b):
        return pltpu.make_async_copy(
            src_hbm.at[pl.ds(e0 + c * _K, _K)], sv[b], si.at[b])

    def dst_copy(c, b):
        return pltpu.make_async_copy(
            dst_hbm.at[pl.ds(e0 + c * _K, _K)], dv[b], si.at[b])

    src_copy(0, 0).start()
    dst_copy(0, 0).start()
    src_copy(1, 1).start()
    dst_copy(1, 1).start()

    def body(p, carry):
        for b in range(2):
            c = 2 * p + b
            src_copy(c, b).wait()
            dst_copy(c, b).wait()
            pltpu.make_async_copy(h_hbm.at[sv[b]], rows_v, sg.at[0]).start()
            pltpu.make_async_copy(h_hbm.at[sv[b]], rows_v, sg.at[0]).wait()
            pltpu.sync_copy(rows_v, acc_sh.at[dv[b]], add=True)

            @pl.when(p < _CHUNKS // 2 - 1)
            def _():
                src_copy(c + 2, b).start()
                dst_copy(c + 2, b).start()
        return carry

    lax.fori_loop(0, _CHUNKS // 2, body, 0)

    plsc.subcore_barrier()
    _node_copy(sid, acc_sh, out_hbm, 0, cid * _N)


_agg_call = pl.kernel(
    _agg_body,
    out_type=jax.ShapeDtypeStruct((2 * _N, _H), jnp.float32),
    mesh=_mesh,
    scratch_types=(
        [pltpu.VMEM((_K,), jnp.int32) for _ in range(4)]
        + [pltpu.VMEM((_K, _H), jnp.float32),
           pltpu.VMEM_SHARED((_ACCR, _H), jnp.float32),
           pltpu.SemaphoreType.DMA((2,)),
           pltpu.SemaphoreType.DMA((1,))]
    ),
)

_R = 1000


def _rspec(w):
    return pl.BlockSpec((_R, w), lambda i: (i, 0))


def _fspec(r, c):
    return pl.BlockSpec((r, c), lambda i: (0, 0))


def _dinv_of(d0_ref, d1_ref):
    return lax.rsqrt(d0_ref[:, 0:1] + d1_ref[:, 0:1] + 1.0)


def _scale_body(d0_ref, d1_ref, x_ref, w_ref, o_ref):
    dinv = _dinv_of(d0_ref, d1_ref)
    m = jnp.dot(x_ref[:], w_ref[:], preferred_element_type=jnp.float32,
                precision=lax.Precision.HIGHEST)
    o_ref[:] = m * dinv


_scale_call = pl.pallas_call(
    _scale_body,
    grid=(_N // _R,),
    in_specs=[_rspec(_DW), _rspec(_DW), _rspec(_D), _fspec(_D, _H)],
    out_specs=_rspec(_H),
    out_shape=jax.ShapeDtypeStruct((_N, _H), jnp.float32),
)


def _mid_body(s0_ref, s1_ref, d0_ref, d1_ref, b1_ref, w2_ref, o_ref):
    dinv = _dinv_of(d0_ref, d1_ref)
    h1 = jnp.maximum((s0_ref[:] + s1_ref[:]) * dinv + b1_ref[:], 0.0)
    m = jnp.dot(h1, w2_ref[:], preferred_element_type=jnp.float32,
                precision=lax.Precision.HIGHEST)
    o_ref[:] = m * dinv


_mid_call = pl.pallas_call(
    _mid_body,
    grid=(_N // _R,),
    in_specs=[_rspec(_H), _rspec(_H), _rspec(_DW), _rspec(_DW),
              _fspec(1, _H), _fspec(_H, _H)],
    out_specs=_rspec(_H),
    out_shape=jax.ShapeDtypeStruct((_N, _H), jnp.float32),
)


def _fin_body(t0_ref, t1_ref, d0_ref, d1_ref, b2_ref, wc_ref, bc_ref, o_ref):
    dinv = _dinv_of(d0_ref, d1_ref)
    h2 = (t0_ref[:] + t1_ref[:]) * dinv + b2_ref[:]
    logits = jnp.dot(h2, wc_ref[:], preferred_element_type=jnp.float32,
                     precision=lax.Precision.HIGHEST) + bc_ref[:]
    m = jnp.max(logits, axis=1, keepdims=True)
    lse = jnp.log(jnp.sum(jnp.exp(logits - m), axis=1, keepdims=True)) + m
    o_ref[:] = logits - lse


_fin_call = pl.pallas_call(
    _fin_body,
    grid=(_N // _R,),
    in_specs=[_rspec(_H), _rspec(_H), _rspec(_DW), _rspec(_DW),
              _fspec(1, _H), _fspec(_H, _C), _fspec(1, _C)],
    out_specs=_rspec(_C),
    out_shape=jax.ShapeDtypeStruct((_N, _C), jnp.float32),
)


def kernel(x, edge_index, W1, b1, W2, b2, Wc, bc):
    npad = _EP - _E
    pad_ar = jnp.arange(npad, dtype=jnp.int32)
    src = jnp.concatenate([edge_index[0], pad_ar % _N])
    dst = jnp.concatenate([edge_index[1], _N + pad_ar % _TRASH])
    z = jnp.zeros((_N, _H), jnp.float32)
    ones = jnp.ones((_K, _DW), jnp.float32)
    degs = _deg_call(dst, z, ones)
    d0, d1 = degs[:_N], degs[_N:]

    h1p = _scale_call(d0, d1, x, W1)
    s = _agg_call(h1p, z, src, dst)
    h2p = _mid_call(s[:_N], s[_N:], d0, d1, b1.reshape(1, _H), W2)
    t = _agg_call(h2p, z, src, dst)
    return _fin_call(t[:_N], t[_N:], d0, d1,
                     b2.reshape(1, _H), Wc, bc.reshape(1, _C))


# speedup vs baseline: 2.5774x; 1.0389x over previous
import jax
import jax.numpy as jnp
from jax import lax
from jax.experimental import pallas as pl
from jax.experimental.pallas import tpu as pltpu
from jax.experimental.pallas import tpu_sc as plsc

_N = 10000
_D = 128
_H = 128
_C = 64
_E = 320000

_NC = 2
_NS = 16
_K = 128
_EP = 327680
_EPT = _EP // (_NC * _NS)
_CHUNKS = _EPT // _K
_SLOTS = 5
_GROUPS = _CHUNKS // _SLOTS
_TRASH = 128
_ACCR = _N + _TRASH
_RPT = 624
_TAIL0 = _RPT * _NS
_TAILN = _N - _TAIL0
_DW = 128

_mesh = plsc.VectorSubcoreMesh(core_axis_name="c", subcore_axis_name="s")


def _node_copy(sid, src, dst, src_base, dst_base):
    r0 = sid * _RPT
    pltpu.sync_copy(src.at[pl.ds(src_base + r0, _RPT)],
                    dst.at[pl.ds(dst_base + r0, _RPT)])

    @pl.when(sid == _NS - 1)
    def _():
        pltpu.sync_copy(src.at[pl.ds(src_base + _TAIL0, _TAILN)],
                        dst.at[pl.ds(dst_base + _TAIL0, _TAILN)])


def _deg_body(dst_hbm, z_hbm, ones_hbm, out_hbm, dv0, dv1, ones_v, acc_sh, si):
    cid = lax.axis_index("c")
    sid = lax.axis_index("s")
    e0 = (cid * _NS + sid) * _EPT
    dv = (dv0, dv1)
    pltpu.sync_copy(ones_hbm, ones_v)
    _node_copy(sid, z_hbm, acc_sh, 0, 0)
    plsc.subcore_barrier()

    def idx_copy(c, b):
        return pltpu.make_async_copy(
            dst_hbm.at[pl.ds(e0 + c * _K, _K)], dv[b], si.at[b])

    idx_copy(0, 0).start()
    idx_copy(1, 1).start()

    def body(p, carry):
        for b in range(2):
            c = 2 * p + b
            idx_copy(c, b).wait()
            pltpu.sync_copy(ones_v, acc_sh.at[dv[b]], add=True)

            @pl.when(p < _CHUNKS // 2 - 1)
            def _():
                idx_copy(c + 2, b).start()
        return carry

    lax.fori_loop(0, _CHUNKS // 2, body, 0)
    plsc.subcore_barrier()
    _node_copy(sid, acc_sh, out_hbm, 0, cid * _N)


_deg_call = pl.kernel(
    _deg_body,
    out_type=jax.ShapeDtypeStruct((2 * _N, _DW), jnp.float32),
    mesh=_mesh,
    scratch_types=[
        pltpu.VMEM((_K,), jnp.int32),
        pltpu.VMEM((_K,), jnp.int32),
        pltpu.VMEM((_K, _DW), jnp.float32),
        pltpu.VMEM_SHARED((_ACCR, _DW), jnp.float32),
        pltpu.SemaphoreType.DMA((2,)),
    ],
)


def _agg_body(h_hbm, z_hbm, src_hbm, dst_hbm, out_hbm,
              sv0, sv1, dv0, dv1, rows_v, acc_sh, si, sg):
    cid = lax.axis_index("c")
    sid = lax.axis_index("s")
    e0 = (cid * _NS + sid) * _EPT
    sv = (sv0, sv1)
    dv = (dv0, dv1)

    @pl.when(cid == 0)
    def _():
        _node_copy(sid, h_hbm, acc_sh, 0, 0)

    @pl.when(cid == 1)
    def _():
        _node_copy(sid, z_hbm, acc_sh, 0, 0)

    plsc.subcore_barrier()

    def src_copy(c, b):
        return pltpu.make_async_copy(
            src_hbm.at[pl.ds(e0 + c * _K, _K)], sv[b], si.at[b])

    def dst_copy(c, b):
        return pltpu.make_async_copy(
            dst_hbm.at[pl.ds(e0 + c * _K, _K)], dv[b], si.at[b])

    src_copy(0, 0).start()
    dst_copy(0, 0).start()
    src_copy(1, 1).start()
    dst_copy(1, 1).start()

    def body(p, carry):
        for b in range(2):
            c = 2 * p + b
            src_copy(c, b).wait()
            dst_copy(c, b).wait()
            pltpu.make_async_copy(h_hbm.at[sv[b]], rows_v, sg.at[0]).start()
            pltpu.make_async_copy(h_hbm.at[sv[b]], rows_v, sg.at[0]).wait()
            pltpu.sync_copy(rows_v, acc_sh.at[dv[b]], add=True)

            @pl.when(p < _CHUNKS // 2 - 1)
            def _():
                src_copy(c + 2, b).start()
                dst_copy(c + 2, b).start()
        return carry

    lax.fori_loop(0, _CHUNKS // 2, body, 0)

    plsc.subcore_barrier()
    _node_copy(sid, acc_sh, out_hbm, 0, cid * _N)


_agg_call = pl.kernel(
    _agg_body,
    out_type=jax.ShapeDtypeStruct((2 * _N, _H), jnp.float32),
    mesh=_mesh,
    scratch_types=(
        [pltpu.VMEM((_K,), jnp.int32) for _ in range(4)]
        + [pltpu.VMEM((_K, _H), jnp.float32),
           pltpu.VMEM_SHARED((_ACCR, _H), jnp.float32),
           pltpu.SemaphoreType.DMA((2,)),
           pltpu.SemaphoreType.DMA((1,))]
    ),
)

_R = 1000


def _rspec(w):
    return pl.BlockSpec((_R, w), lambda i: (i, 0))


def _rspec_hi(w):
    return pl.BlockSpec((_R, w), lambda i: (i + _N // _R, 0))


def _fspec(r, c):
    return pl.BlockSpec((r, c), lambda i: (0, 0))


def _dinv_of(d0_ref, d1_ref):
    return lax.rsqrt(d0_ref[:, 0:1] + d1_ref[:, 0:1] + 1.0)


def _scale_body(d0_ref, d1_ref, x_ref, w_ref, o_ref):
    dinv = _dinv_of(d0_ref, d1_ref)
    m = jnp.dot(x_ref[:], w_ref[:], preferred_element_type=jnp.float32,
                precision=lax.Precision.HIGHEST)
    o_ref[:] = m * dinv


_scale_call = pl.pallas_call(
    _scale_body,
    grid=(_N // _R,),
    in_specs=[_rspec(_DW), _rspec_hi(_DW), _rspec(_D), _fspec(_D, _H)],
    out_specs=_rspec(_H),
    out_shape=jax.ShapeDtypeStruct((_N, _H), jnp.float32),
)


def _mid_body(s0_ref, s1_ref, d0_ref, d1_ref, b1_ref, w2_ref, o_ref):
    dinv = _dinv_of(d0_ref, d1_ref)
    h1 = jnp.maximum((s0_ref[:] + s1_ref[:]) * dinv + b1_ref[:], 0.0)
    m = jnp.dot(h1, w2_ref[:], preferred_element_type=jnp.float32,
                precision=lax.Precision.HIGHEST)
    o_ref[:] = m * dinv


_mid_call = pl.pallas_call(
    _mid_body,
    grid=(_N // _R,),
    in_specs=[_rspec(_H), _rspec_hi(_H), _rspec(_DW), _rspec_hi(_DW),
              _fspec(1, _H), _fspec(_H, _H)],
    out_specs=_rspec(_H),
    out_shape=jax.ShapeDtypeStruct((_N, _H), jnp.float32),
)


def _fin_body(t0_ref, t1_ref, d0_ref, d1_ref, b2_ref, wc_ref, bc_ref, o_ref):
    dinv = _dinv_of(d0_ref, d1_ref)
    h2 = (t0_ref[:] + t1_ref[:]) * dinv + b2_ref[:]
    logits = jnp.dot(h2, wc_ref[:], preferred_element_type=jnp.float32,
                     precision=lax.Precision.HIGHEST) + bc_ref[:]
    m = jnp.max(logits, axis=1, keepdims=True)
    lse = jnp.log(jnp.sum(jnp.exp(logits - m), axis=1, keepdims=True)) + m
    o_ref[:] = logits - lse


_fin_call = pl.pallas_call(
    _fin_body,
    grid=(_N // _R,),
    in_specs=[_rspec(_H), _rspec_hi(_H), _rspec(_DW), _rspec_hi(_DW),
              _fspec(1, _H), _fspec(_H, _C), _fspec(1, _C)],
    out_specs=_rspec(_C),
    out_shape=jax.ShapeDtypeStruct((_N, _C), jnp.float32),
)


def kernel(x, edge_index, W1, b1, W2, b2, Wc, bc):
    npad = _EP - _E
    pad_ar = jnp.arange(npad, dtype=jnp.int32)
    src = jnp.concatenate([edge_index[0], pad_ar % _N])
    dst = jnp.concatenate([edge_index[1], _N + pad_ar % _TRASH])
    z = jnp.zeros((_N, _H), jnp.float32)
    ones = jnp.ones((_K, _DW), jnp.float32)
    degs = _deg_call(dst, z, ones)

    h1p = _scale_call(degs, degs, x, W1)
    s = _agg_call(h1p, z, src, dst)
    h2p = _mid_call(s, s, degs, degs, b1.reshape(1, _H), W2)
    t = _agg_call(h2p, z, src, dst)
    return _fin_call(t, t, degs, degs,
                     b2.reshape(1, _H), Wc, bc.reshape(1, _C))


# speedup vs baseline: 2.8016x; 1.0870x over previous
import jax
import jax.numpy as jnp
from jax import lax
from jax.experimental import pallas as pl
from jax.experimental.pallas import tpu as pltpu
from jax.experimental.pallas import tpu_sc as plsc

_N = 10000
_D = 128
_H = 128
_C = 64
_E = 320000

_NC = 2
_NS = 16
_K = 128
_EP = 327680
_EPT = _EP // (_NC * _NS)
_CHUNKS = _EPT // _K
_SLOTS = 5
_GROUPS = _CHUNKS // _SLOTS
_TRASH = 128
_ACCR = _N + _TRASH
_RPT = 624
_TAIL0 = _RPT * _NS
_TAILN = _N - _TAIL0
_DW = 128

_mesh = plsc.VectorSubcoreMesh(core_axis_name="c", subcore_axis_name="s")


def _node_copy(sid, src, dst, src_base, dst_base):
    r0 = sid * _RPT
    pltpu.sync_copy(src.at[pl.ds(src_base + r0, _RPT)],
                    dst.at[pl.ds(dst_base + r0, _RPT)])

    @pl.when(sid == _NS - 1)
    def _():
        pltpu.sync_copy(src.at[pl.ds(src_base + _TAIL0, _TAILN)],
                        dst.at[pl.ds(dst_base + _TAIL0, _TAILN)])


def _deg_body(dst_hbm, z_hbm, ones_hbm, out_hbm, dv0, dv1, ones_v, acc_sh, si):
    cid = lax.axis_index("c")
    sid = lax.axis_index("s")
    e0 = (cid * _NS + sid) * _EPT
    dv = (dv0, dv1)
    pltpu.sync_copy(ones_hbm, ones_v)
    _node_copy(sid, z_hbm, acc_sh, 0, 0)
    plsc.subcore_barrier()

    def idx_copy(c, b):
        return pltpu.make_async_copy(
            dst_hbm.at[pl.ds(e0 + c * _K, _K)], dv[b], si.at[b])

    idx_copy(0, 0).start()
    idx_copy(1, 1).start()

    def body(p, carry):
        for b in range(2):
            c = 2 * p + b
            idx_copy(c, b).wait()
            pltpu.sync_copy(ones_v, acc_sh.at[dv[b]], add=True)

            @pl.when(p < _CHUNKS // 2 - 1)
            def _():
                idx_copy(c + 2, b).start()
        return carry

    lax.fori_loop(0, _CHUNKS // 2, body, 0)
    plsc.subcore_barrier()
    _node_copy(sid, acc_sh, out_hbm, 0, cid * _N)


_deg_call = pl.kernel(
    _deg_body,
    out_type=jax.ShapeDtypeStruct((2 * _N, _DW), jnp.float32),
    mesh=_mesh,
    scratch_types=[
        pltpu.VMEM((_K,), jnp.int32),
        pltpu.VMEM((_K,), jnp.int32),
        pltpu.VMEM((_K, _DW), jnp.float32),
        pltpu.VMEM_SHARED((_ACCR, _DW), jnp.float32),
        pltpu.SemaphoreType.DMA((2,)),
    ],
)


def _agg_body(h_hbm, z_hbm, src_hbm, dst_hbm, out_hbm,
              sv0, sv1, dv0, dv1, g0, g1, rows_v, acc_sh, si, sg):
    cid = lax.axis_index("c")
    sid = lax.axis_index("s")
    e0 = (cid * _NS + sid) * _EPT
    sv = (sv0, sv1)
    dv = (dv0, dv1)

    @pl.when(cid == 0)
    def _():
        _node_copy(sid, h_hbm, acc_sh, 0, 0)

    @pl.when(cid == 1)
    def _():
        _node_copy(sid, z_hbm, acc_sh, 0, 0)

    plsc.subcore_barrier()

    def src_copy(c, b):
        return pltpu.make_async_copy(
            src_hbm.at[pl.ds(e0 + c * _K, _K)], sv[b], si.at[b])

    def dst_copy(c, b):
        return pltpu.make_async_copy(
            dst_hbm.at[pl.ds(e0 + c * _K, _K)], dv[b], si.at[b])

    gv = (g0, g1)

    def gather(b):
        return pltpu.make_async_copy(h_hbm.at[sv[b]], gv[b], sg.at[b])

    def vcopy(src_ref):
        def row(i, carry):
            for u in range(4):
                for jj in range(8):
                    rows_v[i * 4 + u, pl.ds(jj * 16, 16)] = (
                        src_ref[i * 4 + u, pl.ds(jj * 16, 16)])
            return carry
        lax.fori_loop(0, _K // 4, row, 0)

    src_copy(0, 0).start()
    dst_copy(0, 0).start()
    src_copy(1, 1).start()
    dst_copy(1, 1).start()
    src_copy(0, 0).wait()
    dst_copy(0, 0).wait()
    gather(0).start()

    def body(p, carry):
        for b in range(2):
            c = 2 * p + b
            nb = 1 - b
            gather(b).wait()

            def _start_next():
                src_copy(c + 1, nb).wait()
                dst_copy(c + 1, nb).wait()
                gather(nb).start()

            if b == 0:
                _start_next()
            else:
                @pl.when(p < _CHUNKS // 2 - 1)
                def _():
                    _start_next()

            vcopy(gv[b])
            pltpu.sync_copy(rows_v, acc_sh.at[dv[b]], add=True)

            @pl.when(p < _CHUNKS // 2 - 1)
            def _():
                src_copy(c + 2, b).start()
                dst_copy(c + 2, b).start()
        return carry

    lax.fori_loop(0, _CHUNKS // 2, body, 0)

    plsc.subcore_barrier()
    _node_copy(sid, acc_sh, out_hbm, 0, cid * _N)


_agg_call = pl.kernel(
    _agg_body,
    out_type=jax.ShapeDtypeStruct((2 * _N, _H), jnp.float32),
    mesh=_mesh,
    scratch_types=(
        [pltpu.VMEM((_K,), jnp.int32) for _ in range(4)]
        + [pltpu.VMEM((_K, _H), jnp.float32) for _ in range(3)]
        + [pltpu.VMEM_SHARED((_ACCR, _H), jnp.float32),
           pltpu.SemaphoreType.DMA((2,)),
           pltpu.SemaphoreType.DMA((2,))]
    ),
)

_R = 1000


def _rspec(w):
    return pl.BlockSpec((_R, w), lambda i: (i, 0))


def _rspec_hi(w):
    return pl.BlockSpec((_R, w), lambda i: (i + _N // _R, 0))


def _fspec(r, c):
    return pl.BlockSpec((r, c), lambda i: (0, 0))


def _dinv_of(d0_ref, d1_ref):
    return lax.rsqrt(d0_ref[:, 0:1] + d1_ref[:, 0:1] + 1.0)


def _scale_body(d0_ref, d1_ref, x_ref, w_ref, o_ref):
    dinv = _dinv_of(d0_ref, d1_ref)
    m = jnp.dot(x_ref[:], w_ref[:], preferred_element_type=jnp.float32,
                precision=lax.Precision.HIGHEST)
    o_ref[:] = m * dinv


_scale_call = pl.pallas_call(
    _scale_body,
    grid=(_N // _R,),
    in_specs=[_rspec(_DW), _rspec_hi(_DW), _rspec(_D), _fspec(_D, _H)],
    out_specs=_rspec(_H),
    out_shape=jax.ShapeDtypeStruct((_N, _H), jnp.float32),
)


def _mid_body(s0_ref, s1_ref, d0_ref, d1_ref, b1_ref, w2_ref, o_ref):
    dinv = _dinv_of(d0_ref, d1_ref)
    h1 = jnp.maximum((s0_ref[:] + s1_ref[:]) * dinv + b1_ref[:], 0.0)
    m = jnp.dot(h1, w2_ref[:], preferred_element_type=jnp.float32,
                precision=lax.Precision.HIGHEST)
    o_ref[:] = m * dinv


_mid_call = pl.pallas_call(
    _mid_body,
    grid=(_N // _R,),
    in_specs=[_rspec(_H), _rspec_hi(_H), _rspec(_DW), _rspec_hi(_DW),
              _fspec(1, _H), _fspec(_H, _H)],
    out_specs=_rspec(_H),
    out_shape=jax.ShapeDtypeStruct((_N, _H), jnp.float32),
)


def _fin_body(t0_ref, t1_ref, d0_ref, d1_ref, b2_ref, wc_ref, bc_ref, o_ref):
    dinv = _dinv_of(d0_ref, d1_ref)
    h2 = (t0_ref[:] + t1_ref[:]) * dinv + b2_ref[:]
    logits = jnp.dot(h2, wc_ref[:], preferred_element_type=jnp.float32,
                     precision=lax.Precision.HIGHEST) + bc_ref[:]
    m = jnp.max(logits, axis=1, keepdims=True)
    lse = jnp.log(jnp.sum(jnp.exp(logits - m), axis=1, keepdims=True)) + m
    o_ref[:] = logits - lse


_fin_call = pl.pallas_call(
    _fin_body,
    grid=(_N // _R,),
    in_specs=[_rspec(_H), _rspec_hi(_H), _rspec(_DW), _rspec_hi(_DW),
              _fspec(1, _H), _fspec(_H, _C), _fspec(1, _C)],
    out_specs=_rspec(_C),
    out_shape=jax.ShapeDtypeStruct((_N, _C), jnp.float32),
)


def kernel(x, edge_index, W1, b1, W2, b2, Wc, bc):
    npad = _EP - _E
    pad_ar = jnp.arange(npad, dtype=jnp.int32)
    src = jnp.concatenate([edge_index[0], pad_ar % _N])
    dst = jnp.concatenate([edge_index[1], _N + pad_ar % _TRASH])
    z = jnp.zeros((_N, _H), jnp.float32)
    ones = jnp.ones((_K, _DW), jnp.float32)
    degs = _deg_call(dst, z, ones)

    h1p = _scale_call(degs, degs, x, W1)
    s = _agg_call(h1p, z, src, dst)
    h2p = _mid_call(s, s, degs, degs, b1.reshape(1, _H), W2)
    t = _agg_call(h2p, z, src, dst)
    return _fin_call(t, t, degs, degs,
                     b2.reshape(1, _H), Wc, bc.reshape(1, _C))


# speedup vs baseline: 2.8084x; 1.0024x over previous
import jax
import jax.numpy as jnp
from jax import lax
from jax.experimental import pallas as pl
from jax.experimental.pallas import tpu as pltpu
from jax.experimental.pallas import tpu_sc as plsc

_N = 10000
_D = 128
_H = 128
_C = 64
_E = 320000

_NC = 2
_NS = 16
_K = 128
_EP = 327680
_EPT = _EP // (_NC * _NS)
_CHUNKS = _EPT // _K
_SLOTS = 5
_GROUPS = _CHUNKS // _SLOTS
_TRASH = 128
_ACCR = _N + _TRASH
_RPT = 624
_TAIL0 = _RPT * _NS
_TAILN = _N - _TAIL0
_DW = 128

_mesh = plsc.VectorSubcoreMesh(core_axis_name="c", subcore_axis_name="s")


def _node_copy(sid, src, dst, src_base, dst_base):
    r0 = sid * _RPT
    pltpu.sync_copy(src.at[pl.ds(src_base + r0, _RPT)],
                    dst.at[pl.ds(dst_base + r0, _RPT)])

    @pl.when(sid == _NS - 1)
    def _():
        pltpu.sync_copy(src.at[pl.ds(src_base + _TAIL0, _TAILN)],
                        dst.at[pl.ds(dst_base + _TAIL0, _TAILN)])


def _deg_body(dst_hbm, z_hbm, ones_hbm, out_hbm, dv0, dv1, ones_v, acc_sh, si):
    cid = lax.axis_index("c")
    sid = lax.axis_index("s")
    e0 = (cid * _NS + sid) * _EPT
    dv = (dv0, dv1)
    pltpu.sync_copy(ones_hbm, ones_v)
    _node_copy(sid, z_hbm, acc_sh, 0, 0)
    plsc.subcore_barrier()

    def idx_copy(c, b):
        return pltpu.make_async_copy(
            dst_hbm.at[pl.ds(e0 + c * _K, _K)], dv[b], si.at[b])

    idx_copy(0, 0).start()
    idx_copy(1, 1).start()

    def body(p, carry):
        for b in range(2):
            c = 2 * p + b
            idx_copy(c, b).wait()
            pltpu.sync_copy(ones_v, acc_sh.at[dv[b]], add=True)

            @pl.when(p < _CHUNKS // 2 - 1)
            def _():
                idx_copy(c + 2, b).start()
        return carry

    lax.fori_loop(0, _CHUNKS // 2, body, 0)
    plsc.subcore_barrier()
    _node_copy(sid, acc_sh, out_hbm, 0, cid * _N)


_deg_call = pl.kernel(
    _deg_body,
    out_type=jax.ShapeDtypeStruct((2 * _N, _DW), jnp.float32),
    mesh=_mesh,
    scratch_types=[
        pltpu.VMEM((_K,), jnp.int32),
        pltpu.VMEM((_K,), jnp.int32),
        pltpu.VMEM((_K, _DW), jnp.float32),
        pltpu.VMEM_SHARED((_ACCR, _DW), jnp.float32),
        pltpu.SemaphoreType.DMA((2,)),
    ],
)


def _agg_body(h_hbm, z_hbm, src_hbm, dst_hbm, out_hbm,
              sv0, sv1, dv0, dv1, g0, g1, rows_v, acc_sh, si, sg):
    cid = lax.axis_index("c")
    sid = lax.axis_index("s")
    e0 = (cid * _NS + sid) * _EPT
    sv = (sv0, sv1)
    dv = (dv0, dv1)

    @pl.when(cid == 0)
    def _():
        _node_copy(sid, h_hbm, acc_sh, 0, 0)

    @pl.when(cid == 1)
    def _():
        _node_copy(sid, z_hbm, acc_sh, 0, 0)

    plsc.subcore_barrier()

    def src_copy(c, b):
        return pltpu.make_async_copy(
            src_hbm.at[pl.ds(e0 + c * _K, _K)], sv[b], si.at[b])

    def dst_copy(c, b):
        return pltpu.make_async_copy(
            dst_hbm.at[pl.ds(e0 + c * _K, _K)], dv[b], si.at[b])

    gv = (g0, g1)

    def gather(b):
        return pltpu.make_async_copy(h_hbm.at[sv[b]], gv[b], sg.at[b])

    def vcopy(src_ref):
        def row(i, carry):
            for u in range(4):
                for jj in range(8):
                    rows_v[i * 4 + u, pl.ds(jj * 16, 16)] = (
                        src_ref[i * 4 + u, pl.ds(jj * 16, 16)])
            return carry
        lax.fori_loop(0, _K // 4, row, 0)

    src_copy(0, 0).start()
    dst_copy(0, 0).start()
    src_copy(1, 1).start()
    dst_copy(1, 1).start()
    src_copy(0, 0).wait()
    dst_copy(0, 0).wait()
    gather(0).start()

    def body(p, carry):
        for b in range(2):
            c = 2 * p + b
            nb = 1 - b
            gather(b).wait()

            def _start_next():
                src_copy(c + 1, nb).wait()
                dst_copy(c + 1, nb).wait()
                gather(nb).start()

            if b == 0:
                _start_next()
            else:
                @pl.when(p < _CHUNKS // 2 - 1)
                def _():
                    _start_next()

            vcopy(gv[b])
            pltpu.sync_copy(rows_v, acc_sh.at[dv[b]], add=True)

            @pl.when(p < _CHUNKS // 2 - 1)
            def _():
                src_copy(c + 2, b).start()
                dst_copy(c + 2, b).start()
        return carry

    lax.fori_loop(0, _CHUNKS // 2, body, 0)

    plsc.subcore_barrier()
    _node_copy(sid, acc_sh, out_hbm, 0, cid * _N)


_agg_call = pl.kernel(
    _agg_body,
    out_type=jax.ShapeDtypeStruct((2 * _N, _H), jnp.float32),
    mesh=_mesh,
    scratch_types=(
        [pltpu.VMEM((_K,), jnp.int32) for _ in range(4)]
        + [pltpu.VMEM((_K, _H), jnp.float32) for _ in range(3)]
        + [pltpu.VMEM_SHARED((_ACCR, _H), jnp.float32),
           pltpu.SemaphoreType.DMA((2,)),
           pltpu.SemaphoreType.DMA((2,))]
    ),
)

_R = 1000


def _rspec(w):
    return pl.BlockSpec((_R, w), lambda i: (i, 0))


def _rspec_hi(w):
    return pl.BlockSpec((_R, w), lambda i: (i + _N // _R, 0))


def _fspec(r, c):
    return pl.BlockSpec((r, c), lambda i: (0, 0))


def _dinv_of(d0_ref, d1_ref):
    return lax.rsqrt(d0_ref[:, 0:1] + d1_ref[:, 0:1] + 1.0)


def _scale_body(d0_ref, d1_ref, x_ref, w_ref, o_ref, dv_ref):
    dinv = _dinv_of(d0_ref, d1_ref)
    m = jnp.dot(x_ref[:], w_ref[:], preferred_element_type=jnp.float32,
                precision=lax.Precision.HIGHEST)
    o_ref[:] = m * dinv
    dv_ref[:] = jnp.broadcast_to(dinv, (_R, 8))


_scale_call = pl.pallas_call(
    _scale_body,
    grid=(_N // _R,),
    in_specs=[_rspec(_DW), _rspec_hi(_DW), _rspec(_D), _fspec(_D, _H)],
    out_specs=[_rspec(_H), _rspec(8)],
    out_shape=[jax.ShapeDtypeStruct((_N, _H), jnp.float32),
               jax.ShapeDtypeStruct((_N, 8), jnp.float32)],
)


def _mid_body(s0_ref, s1_ref, dv_ref, b1_ref, w2_ref, o_ref):
    dinv = dv_ref[:, 0:1]
    h1 = jnp.maximum((s0_ref[:] + s1_ref[:]) * dinv + b1_ref[:], 0.0)
    m = jnp.dot(h1, w2_ref[:], preferred_element_type=jnp.float32,
                precision=lax.Precision.HIGHEST)
    o_ref[:] = m * dinv


_mid_call = pl.pallas_call(
    _mid_body,
    grid=(_N // _R,),
    in_specs=[_rspec(_H), _rspec_hi(_H), _rspec(8),
              _fspec(1, _H), _fspec(_H, _H)],
    out_specs=_rspec(_H),
    out_shape=jax.ShapeDtypeStruct((_N, _H), jnp.float32),
)


def _fin_body(t0_ref, t1_ref, dv_ref, b2_ref, wc_ref, bc_ref, o_ref):
    dinv = dv_ref[:, 0:1]
    h2 = (t0_ref[:] + t1_ref[:]) * dinv + b2_ref[:]
    logits = jnp.dot(h2, wc_ref[:], preferred_element_type=jnp.float32,
                     precision=lax.Precision.HIGHEST) + bc_ref[:]
    m = jnp.max(logits, axis=1, keepdims=True)
    lse = jnp.log(jnp.sum(jnp.exp(logits - m), axis=1, keepdims=True)) + m
    o_ref[:] = logits - lse


_fin_call = pl.pallas_call(
    _fin_body,
    grid=(_N // _R,),
    in_specs=[_rspec(_H), _rspec_hi(_H), _rspec(8),
              _fspec(1, _H), _fspec(_H, _C), _fspec(1, _C)],
    out_specs=_rspec(_C),
    out_shape=jax.ShapeDtypeStruct((_N, _C), jnp.float32),
)


def kernel(x, edge_index, W1, b1, W2, b2, Wc, bc):
    npad = _EP - _E
    pad_ar = jnp.arange(npad, dtype=jnp.int32)
    src = jnp.concatenate([edge_index[0], pad_ar % _N])
    dst = jnp.concatenate([edge_index[1], _N + pad_ar % _TRASH])
    z = jnp.zeros((_N, _H), jnp.float32)
    ones = jnp.ones((_K, _DW), jnp.float32)
    degs = _deg_call(dst, z, ones)

    h1p, dinvw = _scale_call(degs, degs, x, W1)
    s = _agg_call(h1p, z, src, dst)
    h2p = _mid_call(s, s, dinvw, b1.reshape(1, _H), W2)
    t = _agg_call(h2p, z, src, dst)
    return _fin_call(t, t, dinvw,
                     b2.reshape(1, _H), Wc, bc.reshape(1, _C))


# speedup vs baseline: 2.8333x; 1.0089x over previous
import jax
import jax.numpy as jnp
from jax import lax
from jax.experimental import pallas as pl
from jax.experimental.pallas import tpu as pltpu
from jax.experimental.pallas import tpu_sc as plsc

_N = 10000
_D = 128
_H = 128
_C = 64
_E = 320000

_NC = 2
_NS = 16
_K = 128
_EP = 327680
_EPT = _EP // (_NC * _NS)
_CHUNKS = _EPT // _K
_SLOTS = 5
_GROUPS = _CHUNKS // _SLOTS
_TRASH = 128
_ACCR = _N + _TRASH
_RPT = 624
_TAIL0 = _RPT * _NS
_TAILN = _N - _TAIL0
_DW = 128

_mesh = plsc.VectorSubcoreMesh(core_axis_name="c", subcore_axis_name="s")


def _node_copy(sid, src, dst, src_base, dst_base):
    r0 = sid * _RPT
    pltpu.sync_copy(src.at[pl.ds(src_base + r0, _RPT)],
                    dst.at[pl.ds(dst_base + r0, _RPT)])

    @pl.when(sid == _NS - 1)
    def _():
        pltpu.sync_copy(src.at[pl.ds(src_base + _TAIL0, _TAILN)],
                        dst.at[pl.ds(dst_base + _TAIL0, _TAILN)])


def _fill_rows(sid, buf, dst):
    r0 = sid * _RPT
    for t in range(_RPT // _K):
        pltpu.sync_copy(buf, dst.at[pl.ds(r0 + t * _K, _K)])
    rem = _RPT % _K
    pltpu.sync_copy(buf.at[pl.ds(0, rem)],
                    dst.at[pl.ds(r0 + (_RPT // _K) * _K, rem)])

    @pl.when(sid == _NS - 1)
    def _():
        pltpu.sync_copy(buf.at[pl.ds(0, _TAILN)],
                        dst.at[pl.ds(_TAIL0, _TAILN)])


def _zero_buf(buf, rows):
    zero = jnp.zeros((16,), jnp.float32)

    def row(i, carry):
        for jj in range(8):
            buf[i, pl.ds(jj * 16, 16)] = zero
        return carry

    lax.fori_loop(0, rows, row, 0)


def _deg_body(dst_hbm, ones_hbm, out_hbm, dv0, dv1, ones_v, zv, acc_sh, si):
    cid = lax.axis_index("c")
    sid = lax.axis_index("s")
    e0 = (cid * _NS + sid) * _EPT
    dv = (dv0, dv1)
    pltpu.sync_copy(ones_hbm, ones_v)
    _zero_buf(zv, _K)
    _fill_rows(sid, zv, acc_sh)
    plsc.subcore_barrier()

    def idx_copy(c, b):
        return pltpu.make_async_copy(
            dst_hbm.at[pl.ds(e0 + c * _K, _K)], dv[b], si.at[b])

    idx_copy(0, 0).start()
    idx_copy(1, 1).start()

    def body(p, carry):
        for b in range(2):
            c = 2 * p + b
            idx_copy(c, b).wait()
            pltpu.sync_copy(ones_v, acc_sh.at[dv[b]], add=True)

            @pl.when(p < _CHUNKS // 2 - 1)
            def _():
                idx_copy(c + 2, b).start()
        return carry

    lax.fori_loop(0, _CHUNKS // 2, body, 0)
    plsc.subcore_barrier()
    _node_copy(sid, acc_sh, out_hbm, 0, cid * _N)


_deg_call = pl.kernel(
    _deg_body,
    out_type=jax.ShapeDtypeStruct((2 * _N, _DW), jnp.float32),
    mesh=_mesh,
    scratch_types=[
        pltpu.VMEM((_K,), jnp.int32),
        pltpu.VMEM((_K,), jnp.int32),
        pltpu.VMEM((_K, _DW), jnp.float32),
        pltpu.VMEM((_K, _DW), jnp.float32),
        pltpu.VMEM_SHARED((_ACCR, _DW), jnp.float32),
        pltpu.SemaphoreType.DMA((2,)),
    ],
)


def _agg_body(h_hbm, src_hbm, dst_hbm, out_hbm,
              sv0, sv1, dv0, dv1, g0, g1, rows_v, acc_sh, si, sg):
    cid = lax.axis_index("c")
    sid = lax.axis_index("s")
    e0 = (cid * _NS + sid) * _EPT
    sv = (sv0, sv1)
    dv = (dv0, dv1)

    @pl.when(cid == 0)
    def _():
        _node_copy(sid, h_hbm, acc_sh, 0, 0)

    @pl.when(cid == 1)
    def _():
        _zero_buf(g0, _K)
        _fill_rows(sid, g0, acc_sh)

    plsc.subcore_barrier()

    def src_copy(c, b):
        return pltpu.make_async_copy(
            src_hbm.at[pl.ds(e0 + c * _K, _K)], sv[b], si.at[b])

    def dst_copy(c, b):
        return pltpu.make_async_copy(
            dst_hbm.at[pl.ds(e0 + c * _K, _K)], dv[b], si.at[b])

    gv = (g0, g1)

    def gather(b):
        return pltpu.make_async_copy(h_hbm.at[sv[b]], gv[b], sg.at[b])

    def vcopy(src_ref):
        def row(i, carry):
            for u in range(4):
                for jj in range(8):
                    rows_v[i * 4 + u, pl.ds(jj * 16, 16)] = (
                        src_ref[i * 4 + u, pl.ds(jj * 16, 16)])
            return carry
        lax.fori_loop(0, _K // 4, row, 0)

    src_copy(0, 0).start()
    dst_copy(0, 0).start()
    src_copy(1, 1).start()
    dst_copy(1, 1).start()
    src_copy(0, 0).wait()
    dst_copy(0, 0).wait()
    gather(0).start()

    def body(p, carry):
        for b in range(2):
            c = 2 * p + b
            nb = 1 - b
            gather(b).wait()

            def _start_next():
                src_copy(c + 1, nb).wait()
                dst_copy(c + 1, nb).wait()
                gather(nb).start()

            if b == 0:
                _start_next()
            else:
                @pl.when(p < _CHUNKS // 2 - 1)
                def _():
                    _start_next()

            vcopy(gv[b])
            pltpu.sync_copy(rows_v, acc_sh.at[dv[b]], add=True)

            @pl.when(p < _CHUNKS // 2 - 1)
            def _():
                src_copy(c + 2, b).start()
                dst_copy(c + 2, b).start()
        return carry

    lax.fori_loop(0, _CHUNKS // 2, body, 0)

    plsc.subcore_barrier()
    _node_copy(sid, acc_sh, out_hbm, 0, cid * _N)


_agg_call = pl.kernel(
    _agg_body,
    out_type=jax.ShapeDtypeStruct((2 * _N, _H), jnp.float32),
    mesh=_mesh,
    scratch_types=(
        [pltpu.VMEM((_K,), jnp.int32) for _ in range(4)]
        + [pltpu.VMEM((_K, _H), jnp.float32) for _ in range(3)]
        + [pltpu.VMEM_SHARED((_ACCR, _H), jnp.float32),
           pltpu.SemaphoreType.DMA((2,)),
           pltpu.SemaphoreType.DMA((2,))]
    ),
)

_R = 1000


def _rspec(w):
    return pl.BlockSpec((_R, w), lambda i: (i, 0))


def _rspec_hi(w):
    return pl.BlockSpec((_R, w), lambda i: (i + _N // _R, 0))


def _fspec(r, c):
    return pl.BlockSpec((r, c), lambda i: (0, 0))


def _dinv_of(d0_ref, d1_ref):
    return lax.rsqrt(d0_ref[:, 0:1] + d1_ref[:, 0:1] + 1.0)


def _scale_body(d0_ref, d1_ref, x_ref, w_ref, o_ref, dv_ref):
    dinv = _dinv_of(d0_ref, d1_ref)
    m = jnp.dot(x_ref[:], w_ref[:], preferred_element_type=jnp.float32,
                precision=lax.Precision.HIGHEST)
    o_ref[:] = m * dinv
    dv_ref[:] = jnp.broadcast_to(dinv, (_R, 8))


_scale_call = pl.pallas_call(
    _scale_body,
    grid=(_N // _R,),
    in_specs=[_rspec(_DW), _rspec_hi(_DW), _rspec(_D), _fspec(_D, _H)],
    out_specs=[_rspec(_H), _rspec(8)],
    out_shape=[jax.ShapeDtypeStruct((_N, _H), jnp.float32),
               jax.ShapeDtypeStruct((_N, 8), jnp.float32)],
)


def _mid_body(s0_ref, s1_ref, dv_ref, b1_ref, w2_ref, o_ref):
    dinv = dv_ref[:, 0:1]
    h1 = jnp.maximum((s0_ref[:] + s1_ref[:]) * dinv + b1_ref[:], 0.0)
    m = jnp.dot(h1, w2_ref[:], preferred_element_type=jnp.float32,
                precision=lax.Precision.HIGHEST)
    o_ref[:] = m * dinv


_mid_call = pl.pallas_call(
    _mid_body,
    grid=(_N // _R,),
    in_specs=[_rspec(_H), _rspec_hi(_H), _rspec(8),
              _fspec(1, _H), _fspec(_H, _H)],
    out_specs=_rspec(_H),
    out_shape=jax.ShapeDtypeStruct((_N, _H), jnp.float32),
)


def _fin_body(t0_ref, t1_ref, dv_ref, b2_ref, wc_ref, bc_ref, o_ref):
    dinv = dv_ref[:, 0:1]
    h2 = (t0_ref[:] + t1_ref[:]) * dinv + b2_ref[:]
    logits = jnp.dot(h2, wc_ref[:], preferred_element_type=jnp.float32,
                     precision=lax.Precision.HIGHEST) + bc_ref[:]
    m = jnp.max(logits, axis=1, keepdims=True)
    lse = jnp.log(jnp.sum(jnp.exp(logits - m), axis=1, keepdims=True)) + m
    o_ref[:] = logits - lse


_fin_call = pl.pallas_call(
    _fin_body,
    grid=(_N // _R,),
    in_specs=[_rspec(_H), _rspec_hi(_H), _rspec(8),
              _fspec(1, _H), _fspec(_H, _C), _fspec(1, _C)],
    out_specs=_rspec(_C),
    out_shape=jax.ShapeDtypeStruct((_N, _C), jnp.float32),
)


def kernel(x, edge_index, W1, b1, W2, b2, Wc, bc):
    npad = _EP - _E
    pad_ar = jnp.arange(npad, dtype=jnp.int32)
    src = jnp.concatenate([edge_index[0], pad_ar % _N])
    dst = jnp.concatenate([edge_index[1], _N + pad_ar % _TRASH])
    ones = jnp.ones((_K, _DW), jnp.float32)
    degs = _deg_call(dst, ones)

    h1p, dinvw = _scale_call(degs, degs, x, W1)
    s = _agg_call(h1p, src, dst)
    h2p = _mid_call(s, s, dinvw, b1.reshape(1, _H), W2)
    t = _agg_call(h2p, src, dst)
    return _fin_call(t, t, dinvw,
                     b2.reshape(1, _H), Wc, bc.reshape(1, _C))


# speedup vs baseline: 2.8346x; 1.0005x over previous
import jax
import jax.numpy as jnp
from jax import lax
from jax.experimental import pallas as pl
from jax.experimental.pallas import tpu as pltpu
from jax.experimental.pallas import tpu_sc as plsc

_N = 10000
_D = 128
_H = 128
_C = 64
_E = 320000

_NC = 2
_NS = 16
_K = 128
_EP = 327680
_EPT = _EP // (_NC * _NS)
_CHUNKS = _EPT // _K
_TRASH = 128
_ACCR = _N + _TRASH
_RPT = 624
_TAIL0 = _RPT * _NS
_TAILN = _N - _TAIL0
_DW = 128

_mesh = plsc.VectorSubcoreMesh(core_axis_name="c", subcore_axis_name="s")


def _node_copy(sid, src, dst, src_base, dst_base):
    r0 = sid * _RPT
    pltpu.sync_copy(src.at[pl.ds(src_base + r0, _RPT)],
                    dst.at[pl.ds(dst_base + r0, _RPT)])

    @pl.when(sid == _NS - 1)
    def _():
        pltpu.sync_copy(src.at[pl.ds(src_base + _TAIL0, _TAILN)],
                        dst.at[pl.ds(dst_base + _TAIL0, _TAILN)])


def _fill_rows(sid, buf, dst):
    r0 = sid * _RPT
    for t in range(_RPT // _K):
        pltpu.sync_copy(buf, dst.at[pl.ds(r0 + t * _K, _K)])
    rem = _RPT % _K
    pltpu.sync_copy(buf.at[pl.ds(0, rem)],
                    dst.at[pl.ds(r0 + (_RPT // _K) * _K, rem)])

    @pl.when(sid == _NS - 1)
    def _():
        pltpu.sync_copy(buf.at[pl.ds(0, _TAILN)],
                        dst.at[pl.ds(_TAIL0, _TAILN)])


def _zero_buf(buf, rows):
    zero = jnp.zeros((16,), jnp.float32)

    def row(i, carry):
        for jj in range(8):
            buf[i, pl.ds(jj * 16, 16)] = zero
        return carry

    lax.fori_loop(0, rows, row, 0)


def _deg_body(dst_hbm, ones_hbm, out_hbm, dv0, dv1, ones_v, zv, acc_sh, si):
    cid = lax.axis_index("c")
    sid = lax.axis_index("s")
    e0 = (cid * _NS + sid) * _EPT
    dv = (dv0, dv1)
    pltpu.sync_copy(ones_hbm, ones_v)
    _zero_buf(zv, _K)
    _fill_rows(sid, zv, acc_sh)
    plsc.subcore_barrier()

    def idx_copy(c, b):
        return pltpu.make_async_copy(
            dst_hbm.at[pl.ds(e0 + c * _K, _K)], dv[b], si.at[b])

    idx_copy(0, 0).start()
    idx_copy(1, 1).start()

    def body(p, carry):
        for b in range(2):
            c = 2 * p + b
            idx_copy(c, b).wait()
            pltpu.sync_copy(ones_v, acc_sh.at[dv[b]], add=True)

            @pl.when(p < _CHUNKS // 2 - 1)
            def _():
                idx_copy(c + 2, b).start()
        return carry

    lax.fori_loop(0, _CHUNKS // 2, body, 0)
    plsc.subcore_barrier()
    _node_copy(sid, acc_sh, out_hbm, 0, cid * _N)


_deg_call = pl.kernel(
    _deg_body,
    out_type=jax.ShapeDtypeStruct((2 * _N, _DW), jnp.float32),
    mesh=_mesh,
    scratch_types=[
        pltpu.VMEM((_K,), jnp.int32),
        pltpu.VMEM((_K,), jnp.int32),
        pltpu.VMEM((_K, _DW), jnp.float32),
        pltpu.VMEM((_K, _DW), jnp.float32),
        pltpu.VMEM_SHARED((_ACCR, _DW), jnp.float32),
        pltpu.SemaphoreType.DMA((2,)),
    ],
)


def _agg_body(h_hbm, src_hbm, dst_hbm, out_hbm,
              sv0, sv1, dv0, dv1, g0, g1, rows_v, acc_sh, si, sg):
    cid = lax.axis_index("c")
    sid = lax.axis_index("s")
    e0 = (cid * _NS + sid) * _EPT
    sv = (sv0, sv1)
    dv = (dv0, dv1)

    @pl.when(cid == 0)
    def _():
        _node_copy(sid, h_hbm, acc_sh, 0, 0)

    @pl.when(cid == 1)
    def _():
        _zero_buf(g0, _K)
        _fill_rows(sid, g0, acc_sh)

    plsc.subcore_barrier()

    def src_copy(c, b):
        return pltpu.make_async_copy(
            src_hbm.at[pl.ds(e0 + c * _K, _K)], sv[b], si.at[b])

    def dst_copy(c, b):
        return pltpu.make_async_copy(
            dst_hbm.at[pl.ds(e0 + c * _K, _K)], dv[b], si.at[b])

    gv = (g0, g1)

    def gather(b):
        return pltpu.make_async_copy(h_hbm.at[sv[b]], gv[b], sg.at[b])

    def vcopy(src_ref):
        def row(i, carry):
            for u in range(4):
                for jj in range(8):
                    rows_v[i * 4 + u, pl.ds(jj * 16, 16)] = (
                        src_ref[i * 4 + u, pl.ds(jj * 16, 16)])
            return carry
        lax.fori_loop(0, _K // 4, row, 0)

    src_copy(0, 0).start()
    dst_copy(0, 0).start()
    src_copy(1, 1).start()
    dst_copy(1, 1).start()
    src_copy(0, 0).wait()
    dst_copy(0, 0).wait()
    gather(0).start()

    def body(p, carry):
        for b in range(2):
            c = 2 * p + b
            nb = 1 - b
            gather(b).wait()

            def _start_next():
                src_copy(c + 1, nb).wait()
                dst_copy(c + 1, nb).wait()
                gather(nb).start()

            if b == 0:
                _start_next()
            else:
                @pl.when(p < _CHUNKS // 2 - 1)
                def _():
                    _start_next()

            vcopy(gv[b])
            pltpu.sync_copy(rows_v, acc_sh.at[dv[b]], add=True)

            @pl.when(p < _CHUNKS // 2 - 1)
            def _():
                src_copy(c + 2, b).start()
                dst_copy(c + 2, b).start()
        return carry

    lax.fori_loop(0, _CHUNKS // 2, body, 0)

    plsc.subcore_barrier()
    _node_copy(sid, acc_sh, out_hbm, 0, cid * _N)


_agg_call = pl.kernel(
    _agg_body,
    out_type=jax.ShapeDtypeStruct((2 * _N, _H), jnp.float32),
    mesh=_mesh,
    scratch_types=(
        [pltpu.VMEM((_K,), jnp.int32) for _ in range(4)]
        + [pltpu.VMEM((_K, _H), jnp.float32) for _ in range(3)]
        + [pltpu.VMEM_SHARED((_ACCR, _H), jnp.float32),
           pltpu.SemaphoreType.DMA((2,)),
           pltpu.SemaphoreType.DMA((2,))]
    ),
)

_R = 1000


def _rspec(w):
    return pl.BlockSpec((_R, w), lambda i: (i, 0))


def _rspec_hi(w):
    return pl.BlockSpec((_R, w), lambda i: (i + _N // _R, 0))


def _fspec(r, c):
    return pl.BlockSpec((r, c), lambda i: (0, 0))


def _dinv_of(d0_ref, d1_ref):
    return lax.rsqrt(d0_ref[:, 0:1] + d1_ref[:, 0:1] + 1.0)


def _scale_body(d0_ref, d1_ref, x_ref, w_ref, o_ref, dv_ref):
    dinv = _dinv_of(d0_ref, d1_ref)
    m = jnp.dot(x_ref[:], w_ref[:], preferred_element_type=jnp.float32,
                precision=lax.Precision.HIGHEST)
    o_ref[:] = m * dinv
    dv_ref[:] = jnp.broadcast_to(dinv, (_R, 8))


_scale_call = pl.pallas_call(
    _scale_body,
    grid=(_N // _R,),
    in_specs=[_rspec(_DW), _rspec_hi(_DW), _rspec(_D), _fspec(_D, _H)],
    out_specs=[_rspec(_H), _rspec(8)],
    out_shape=[jax.ShapeDtypeStruct((_N, _H), jnp.float32),
               jax.ShapeDtypeStruct((_N, 8), jnp.float32)],
)


def _mid_body(s0_ref, s1_ref, dv_ref, b1_ref, w2_ref, o_ref):
    dinv = dv_ref[:, 0:1]
    h1 = jnp.maximum((s0_ref[:] + s1_ref[:]) * dinv + b1_ref[:], 0.0)
    m = jnp.dot(h1, w2_ref[:], preferred_element_type=jnp.float32,
                precision=lax.Precision.HIGHEST)
    o_ref[:] = m * dinv


_mid_call = pl.pallas_call(
    _mid_body,
    grid=(_N // _R,),
    in_specs=[_rspec(_H), _rspec_hi(_H), _rspec(8),
              _fspec(1, _H), _fspec(_H, _H)],
    out_specs=_rspec(_H),
    out_shape=jax.ShapeDtypeStruct((_N, _H), jnp.float32),
)


def _fin_body(t0_ref, t1_ref, dv_ref, b2_ref, wc_ref, bc_ref, o_ref):
    dinv = dv_ref[:, 0:1]
    h2 = (t0_ref[:] + t1_ref[:]) * dinv + b2_ref[:]
    logits = jnp.dot(h2, wc_ref[:], preferred_element_type=jnp.float32,
                     precision=lax.Precision.HIGHEST) + bc_ref[:]
    m = jnp.max(logits, axis=1, keepdims=True)
    lse = jnp.log(jnp.sum(jnp.exp(logits - m), axis=1, keepdims=True)) + m
    o_ref[:] = logits - lse


_fin_call = pl.pallas_call(
    _fin_body,
    grid=(_N // _R,),
    in_specs=[_rspec(_H), _rspec_hi(_H), _rspec(8),
              _fspec(1, _H), _fspec(_H, _C), _fspec(1, _C)],
    out_specs=_rspec(_C),
    out_shape=jax.ShapeDtypeStruct((_N, _C), jnp.float32),
)


def kernel(x, edge_index, W1, b1, W2, b2, Wc, bc):
    npad = _EP - _E
    pad_ar = jnp.arange(npad, dtype=jnp.int32)
    src = jnp.concatenate([edge_index[0], pad_ar % _N])
    dst = jnp.concatenate([edge_index[1], _N + pad_ar % _TRASH])
    ones = jnp.ones((_K, _DW), jnp.float32)
    degs = _deg_call(dst, ones)

    h1p, dinvw = _scale_call(degs, degs, x, W1)
    s = _agg_call(h1p, src, dst)
    h2p = _mid_call(s, s, dinvw, b1.reshape(1, _H), W2)
    t = _agg_call(h2p, src, dst)
    return _fin_call(t, t, dinvw,
                     b2.reshape(1, _H), Wc, bc.reshape(1, _C))
